# baseline (speedup 1.0000x reference)
import functools

import jax
import jax.numpy as jnp
from jax import lax
from jax.experimental import pallas as pl
from jax.experimental.pallas import tpu as pltpu
from jax.experimental.pallas import tpu_sc as plsc

R, C = 16384, 200
V, D = 100, 36
D2 = 2 * D
JP = C // 2
NP = R * JP
G = 128
G2 = 2 * G
NG = NP // G
NC, NS = 2, 16
NW = NC * NS
GPW = NG // NW
K = 4
KG = K * G
NCHUNK = GPW // K

IC = 32
NS2 = D2 // 8
IPW = R // NW
NC2 = IPW // IC
NSTEP = NC2 * NS2


def _sc_gather(x2d, ptab):
    mesh = plsc.VectorSubcoreMesh(core_axis_name="c", subcore_axis_name="s")

    @functools.partial(
        pl.kernel,
        mesh=mesh,
        out_type=jax.ShapeDtypeStruct((NP, D2), jnp.float32),
        scratch_types=[
            pltpu.VMEM((3, K, G2), jnp.int32),
            pltpu.VMEM((3, K, G), jnp.int32),
            pltpu.VMEM((2, KG, D2), jnp.float32),
            pltpu.SemaphoreType.DMA,
            pltpu.SemaphoreType.DMA,
            pltpu.SemaphoreType.DMA,
        ],
        compiler_params=pltpu.CompilerParams(
            use_tc_tiling_on_sc=False, needs_layout_passes=False
        ),
    )
    def k(x_hbm, p_hbm, out_hbm, raw_v, idx_v, rows_v, isem, gsem, ssem):
        wid = lax.axis_index("s") * NC + lax.axis_index("c")
        base = wid * GPW
        last = NCHUNK - 1
        lanes = lax.iota(jnp.int32, 16)

        def fire_idx(b, c):
            pltpu.async_copy(x_hbm.at[pl.ds(base + c * K, K)], raw_v.at[b], isem)

        def wait_idx(b, c):
            pltpu.make_async_copy(
                x_hbm.at[pl.ds(base + c * K, K)], raw_v.at[b], isem
            ).wait()

        def make_pairs(b):
            for j in range(K):
                row = raw_v.at[b, j]
                for t in range(G // 16):
                    pos = lanes * 2 + (t * 32)
                    a = plsc.load_gather(row, [pos])
                    bb = plsc.load_gather(row, [pos + 1])
                    idx_v[b, j, pl.ds(t * 16, 16)] = a * V + bb

        def fire_gathers(bi, br):
            for j in range(K):
                pltpu.async_copy(
                    p_hbm.at[idx_v.at[bi, j]],
                    rows_v.at[br, pl.ds(j * G, G)],
                    gsem,
                )

        def wait_gathers(br, c):
            pltpu.make_async_copy(
                out_hbm.at[pl.ds((base + c * K) * G, KG)], rows_v.at[br], gsem
            ).wait()

        def fire_scatter(br, c):
            pltpu.async_copy(
                rows_v.at[br], out_hbm.at[pl.ds((base + c * K) * G, KG)], ssem
            )

        def wait_scatter(br, c):
            pltpu.make_async_copy(
                rows_v.at[br], out_hbm.at[pl.ds((base + c * K) * G, KG)], ssem
            ).wait()

        fire_idx(0, 0)
        fire_idx(1, 1)
        wait_idx(0, 0)
        make_pairs(0)
        fire_gathers(0, 0)

        def body(i, carry):
            b3 = i % 3
            br = i % 2
            brp = (i - 1) % 2
            wait_idx(b3, i)
            make_pairs(b3)

            @pl.when(i >= 2)
            def _():
                wait_scatter(br, i - 2)

            fire_gathers(b3, br)

            @pl.when(i < last)
            def _():
                fire_idx((i + 1) % 3, i + 1)

            wait_gathers(brp, i - 1)
            fire_scatter(brp, i - 1)
            return carry

        lax.fori_loop(1, NCHUNK, body, 0)

        wait_gathers(last % 2, last)
        fire_scatter(last % 2, last)
        wait_scatter((last - 1) % 2, last - 1)
        wait_scatter(last % 2, last)

    return k(x2d, ptab)


def _sc_permute(rows):
    mesh = plsc.VectorSubcoreMesh(core_axis_name="c", subcore_axis_name="s")

    @functools.partial(
        pl.kernel,
        mesh=mesh,
        out_type=jax.ShapeDtypeStruct((D2, JP, R), jnp.float32),
        scratch_types=[
            pltpu.VMEM((2, IC * JP, 8), jnp.float32),
            pltpu.VMEM((2, 8, JP, IC), jnp.float32),
            pltpu.SemaphoreType.DMA,
            pltpu.SemaphoreType.DMA,
        ],
        compiler_params=pltpu.CompilerParams(
            use_tc_tiling_on_sc=False, needs_layout_passes=False
        ),
    )
    def k(a_hbm, o_hbm, ins_v, outs_v, isem, osem):
        wid = lax.axis_index("s") * NC + lax.axis_index("c")
        ibase = wid * IPW
        lanes = lax.iota(jnp.int32, 16)

        def src_slab(t):
            cc = t // NS2
            s = t % NS2
            r0 = (ibase + cc * IC) * JP
            return a_hbm.at[pl.ds(r0, IC * JP), pl.ds(s * 8, 8)]

        def dst_slab(t):
            cc = t // NS2
            s = t % NS2
            return o_hbm.at[pl.ds(s * 8, 8), :, pl.ds(ibase + cc * IC, IC)]

        def fire_in(t, b):
            pltpu.async_copy(src_slab(t), ins_v.at[b], isem)

        def wait_in(t, b):
            pltpu.make_async_copy(src_slab(t), ins_v.at[b], isem).wait()

        def fire_out(t, b):
            pltpu.async_copy(outs_v.at[b], dst_slab(t), osem)

        def wait_out(t, b):
            pltpu.make_async_copy(outs_v.at[b], dst_slab(t), osem).wait()

        def permute(b):
            src = ins_v.at[b]

            def jp_body(jp, carry):
                for w in range(8):
                    for h in range(IC // 16):
                        rowsel = (lanes + h * 16) * JP + jp
                        v = plsc.load_gather(src, [rowsel, jnp.full((16,), w, jnp.int32)])
                        outs_v[b, w, jp, pl.ds(h * 16, 16)] = v
                return carry

            lax.fori_loop(0, JP, jp_body, 0)

        fire_in(0, 0)
        fire_in(1, 1)

        def body(t, carry):
            b = t % 2
            wait_in(t, b)

            @pl.when(t >= 2)
            def _():
                wait_out(t - 2, b)

            permute(b)
            fire_out(t, b)

            @pl.when(t + 2 < NSTEP)
            def _():
                fire_in(t + 2, b)

            return carry

        lax.fori_loop(0, NSTEP, body, 0)
        wait_out(NSTEP - 2, NSTEP % 2)
        wait_out(NSTEP - 1, (NSTEP - 1) % 2)

    return k(rows)


def kernel(x, wte):
    x2d = x.reshape(NG, G2).astype(jnp.int32)
    ptab = jnp.stack(
        [
            jnp.broadcast_to(wte[:, None, :], (V, V, D)),
            jnp.broadcast_to(wte[None, :, :], (V, V, D)),
        ],
        axis=-1,
    ).reshape(V * V, D2)
    rows = _sc_gather(x2d, ptab)
    out2 = _sc_permute(rows)
    return jnp.transpose(out2.reshape(D, 2, JP, R), (3, 2, 1, 0)).reshape(R, C, D)


# speedup vs baseline: 1.4107x; 1.4107x over previous
import functools

import jax
import jax.numpy as jnp
from jax import lax
from jax.experimental import pallas as pl
from jax.experimental.pallas import tpu as pltpu
from jax.experimental.pallas import tpu_sc as plsc

R, C = 16384, 200
V, D = 100, 36
D2 = 2 * D
NP = R * C // 2
G = 128
G2 = 2 * G
NG = NP // G
NC, NS = 2, 16
NW = NC * NS
GPW = NG // NW
K = 4
NCHUNK = GPW // K


def _sc_gather(x2d, ptab):
    mesh = plsc.VectorSubcoreMesh(core_axis_name="c", subcore_axis_name="s")

    @functools.partial(
        pl.kernel,
        mesh=mesh,
        out_type=jax.ShapeDtypeStruct((NG, G, D2), jnp.float32),
        scratch_types=[
            pltpu.VMEM((3, K, G2), jnp.int32),
            pltpu.VMEM((3, K, G), jnp.int32),
            pltpu.VMEM((2, K, G, D2), jnp.float32),
            pltpu.SemaphoreType.DMA,
            pltpu.SemaphoreType.DMA,
            pltpu.SemaphoreType.DMA,
        ],
        compiler_params=pltpu.CompilerParams(
            use_tc_tiling_on_sc=False, needs_layout_passes=False
        ),
    )
    def k(x_hbm, p_hbm, out_hbm, raw_v, idx_v, rows_v, isem, gsem, ssem):
        wid = lax.axis_index("s") * NC + lax.axis_index("c")
        base = wid * GPW
        last = NCHUNK - 1
        lanes = lax.iota(jnp.int32, 16)

        def fire_idx(b, c):
            pltpu.async_copy(x_hbm.at[pl.ds(base + c * K, K)], raw_v.at[b], isem)

        def wait_idx(b, c):
            pltpu.make_async_copy(
                x_hbm.at[pl.ds(base + c * K, K)], raw_v.at[b], isem
            ).wait()

        def make_pairs(b):
            for j in range(K):
                row = raw_v.at[b, j]
                for t in range(G // 16):
                    pos = lanes * 2 + (t * 32)
                    a = plsc.load_gather(row, [pos])
                    bb = plsc.load_gather(row, [pos + 1])
                    idx_v[b, j, pl.ds(t * 16, 16)] = a * V + bb

        def fire_gathers(bi, br):
            for j in range(K):
                pltpu.async_copy(p_hbm.at[idx_v.at[bi, j]], rows_v.at[br, j], gsem)

        def wait_gathers(br, c):
            pltpu.make_async_copy(
                out_hbm.at[pl.ds(base + c * K, K)], rows_v.at[br], gsem
            ).wait()

        def fire_scatter(br, c):
            pltpu.async_copy(rows_v.at[br], out_hbm.at[pl.ds(base + c * K, K)], ssem)

        def wait_scatter(br, c):
            pltpu.make_async_copy(
                rows_v.at[br], out_hbm.at[pl.ds(base + c * K, K)], ssem
            ).wait()

        fire_idx(0, 0)
        fire_idx(1, 1)
        wait_idx(0, 0)
        make_pairs(0)
        fire_gathers(0, 0)

        def body(i, carry):
            b3 = i % 3
            br = i % 2
            brp = (i - 1) % 2
            wait_idx(b3, i)
            make_pairs(b3)

            @pl.when(i >= 2)
            def _():
                wait_scatter(br, i - 2)

            fire_gathers(b3, br)

            @pl.when(i < last)
            def _():
                fire_idx((i + 1) % 3, i + 1)

            wait_gathers(brp, i - 1)
            fire_scatter(brp, i - 1)
            return carry

        lax.fori_loop(1, NCHUNK, body, 0)

        wait_gathers(last % 2, last)
        fire_scatter(last % 2, last)
        wait_scatter((last - 1) % 2, last - 1)
        wait_scatter(last % 2, last)

    return k(x2d, ptab)


def kernel(x, wte):
    x2d = x.reshape(NG, G2).astype(jnp.int32)
    ptab = jnp.concatenate(
        [
            jnp.broadcast_to(wte[:, None, :], (V, V, D)),
            jnp.broadcast_to(wte[None, :, :], (V, V, D)),
        ],
        axis=-1,
    ).reshape(V * V, D2)
    out = _sc_gather(x2d, ptab)
    return out.reshape(R, C, D)
